# baseline (speedup 1.0000x reference)
import functools

import jax
import jax.numpy as jnp
from jax import lax
from jax.experimental import pallas as pl
from jax.experimental.pallas import tpu as pltpu
from jax.experimental.pallas import tpu_sc as plsc

N_NODES = 10000
N_EDGES = 320000
D = 128

NC = 2
NS = 16
NW = NC * NS
F_PER = D // NW
SEG = F_PER * N_NODES
CHUNK = 8000
NB = CHUNK // 16
NCHUNKS = N_EDGES // CHUNK


def _mm1_body(w_ref, x_ref, b_ref, o_ref):
    acc = lax.dot_general(w_ref[...], x_ref[...], (((1,), (1,)), ((), ())),
                          preferred_element_type=jnp.float32)
    o_ref[...] = jnp.maximum(acc + b_ref[...], 0.0)


def _mm2_body(a_ref, w_ref, b_ref, o_ref):
    acc = lax.dot_general(a_ref[...], w_ref[...], (((0,), (1,)), ((), ())),
                          preferred_element_type=jnp.float32)
    o_ref[...] = acc + b_ref[...]


def _segmax_body(h_hbm, s_hbm, r_hbm, out_hbm, h_v,
                 acc0, acc1, acc2, acc3, s_buf, r_buf):
    accs = (acc0, acc1, acc2, acc3)
    wid = lax.axis_index("s") * NC + lax.axis_index("c")
    base = wid * SEG
    pltpu.sync_copy(h_hbm.at[pl.ds(base, SEG)], h_v)

    def zero_body(i, c):
        for f in range(F_PER):
            accs[f][pl.ds(i * 16, 16)] = jnp.zeros((16,), jnp.float32)
        return c
    lax.fori_loop(0, N_NODES // 16, zero_body, 0)

    def chunk_body(g, c):
        pltpu.sync_copy(s_hbm.at[pl.ds(g * CHUNK, CHUNK)], s_buf)
        pltpu.sync_copy(r_hbm.at[pl.ds(g * CHUNK, CHUNK)], r_buf)

        def batch_body(b, c2):
            s16 = s_buf[pl.ds(b * 16, 16)]
            r16 = r_buf[pl.ds(b * 16, 16)]
            news, pends = [], []
            for f in range(F_PER):
                hv = plsc.load_gather(h_v, [s16 + (f * N_NODES)])
                cur = plsc.load_gather(accs[f], [r16])
                nw = jnp.maximum(cur, hv)
                plsc.store_scatter(accs[f], [r16], nw)
                chk = plsc.load_gather(accs[f], [r16])
                news.append(nw)
                pends.append(chk < nw)

            def cond(ps):
                return jnp.any(ps[0] | ps[1] | ps[2] | ps[3])

            def retry(ps):
                out_ps = []
                for f in range(F_PER):
                    plsc.store_scatter(accs[f], [r16], news[f], mask=ps[f])
                    chk = plsc.load_gather(accs[f], [r16])
                    out_ps.append(ps[f] & (chk < news[f]))
                return tuple(out_ps)

            lax.while_loop(cond, retry, tuple(pends))
            return c2
        lax.fori_loop(0, NB, batch_body, 0)
        return c
    lax.fori_loop(0, NCHUNKS, chunk_body, 0)

    for f in range(F_PER):
        pltpu.sync_copy(accs[f], out_hbm.at[pl.ds(base + f * N_NODES, N_NODES)])


_segmax = functools.partial(
    pl.kernel,
    mesh=plsc.VectorSubcoreMesh(core_axis_name="c", subcore_axis_name="s",
                                num_cores=NC, num_subcores=NS),
    out_type=jax.ShapeDtypeStruct((D * N_NODES,), jnp.float32),
    compiler_params=pltpu.CompilerParams(needs_layout_passes=False),
    scratch_types=[
        pltpu.VMEM((SEG,), jnp.float32),
        pltpu.VMEM((N_NODES,), jnp.float32),
        pltpu.VMEM((N_NODES,), jnp.float32),
        pltpu.VMEM((N_NODES,), jnp.float32),
        pltpu.VMEM((N_NODES,), jnp.float32),
        pltpu.VMEM((CHUNK,), jnp.int32),
        pltpu.VMEM((CHUNK,), jnp.int32),
    ],
)(_segmax_body)


def kernel(node_features, senders, receivers, W1, b1, W2, b2):
    h_T = pl.pallas_call(
        _mm1_body,
        out_shape=jax.ShapeDtypeStruct((D, N_NODES), jnp.float32),
    )(W1, node_features, b1.reshape(D, 1))

    agg_flat = _segmax(h_T.reshape(-1),
                       senders.astype(jnp.int32),
                       receivers.astype(jnp.int32))

    nodes = pl.pallas_call(
        _mm2_body,
        out_shape=jax.ShapeDtypeStruct((N_NODES, D), jnp.float32),
    )(agg_flat.reshape(D, N_NODES), W2, b2.reshape(1, D))
    return nodes


# speedup vs baseline: 1.3929x; 1.3929x over previous
import functools

import jax
import jax.numpy as jnp
from jax import lax
from jax.experimental import pallas as pl
from jax.experimental.pallas import tpu as pltpu
from jax.experimental.pallas import tpu_sc as plsc

N_NODES = 10000
N_EDGES = 320000
D = 128

NC = 2
NS = 16
NW = NC * NS
F_PER = D // NW
SEG = F_PER * N_NODES
CHUNK = 8000
NB = CHUNK // 16
NCHUNKS = N_EDGES // CHUNK


def _mm1_body(w_ref, x_ref, b_ref, o_ref):
    acc = lax.dot_general(w_ref[...], x_ref[...], (((1,), (1,)), ((), ())),
                          preferred_element_type=jnp.float32)
    o_ref[...] = jnp.maximum(acc + b_ref[...], 0.0)


def _mm2_body(a_ref, w_ref, b_ref, o_ref):
    acc = lax.dot_general(a_ref[...], w_ref[...], (((0,), (1,)), ((), ())),
                          preferred_element_type=jnp.float32)
    o_ref[...] = acc + b_ref[...]


def _segmax_body(h_hbm, s_hbm, r_hbm, out_hbm, h_v,
                 acc0, acc1, acc2, acc3, s_buf, r_buf):
    accs = (acc0, acc1, acc2, acc3)
    wid = lax.axis_index("s") * NC + lax.axis_index("c")
    base = wid * SEG
    pltpu.sync_copy(h_hbm.at[pl.ds(base, SEG)], h_v)

    def zero_body(i, c):
        for f in range(F_PER):
            accs[f][pl.ds(i * 16, 16)] = jnp.zeros((16,), jnp.float32)
        return c
    lax.fori_loop(0, N_NODES // 16, zero_body, 0)

    def chunk_body(g, c):
        pltpu.sync_copy(s_hbm.at[pl.ds(g * CHUNK, CHUNK)], s_buf)
        pltpu.sync_copy(r_hbm.at[pl.ds(g * CHUNK, CHUNK)], r_buf)

        def fast_batch(b, unresolved):
            s16 = s_buf[pl.ds(b * 16, 16)]
            r16 = r_buf[pl.ds(b * 16, 16)]
            news, pends = [], []
            for f in range(F_PER):
                hv = plsc.load_gather(h_v, [s16 + (f * N_NODES)])
                cur = plsc.load_gather(accs[f], [r16])
                nw = jnp.maximum(cur, hv)
                plsc.store_scatter(accs[f], [r16], nw)
                chk = plsc.load_gather(accs[f], [r16])
                news.append(nw)
                pends.append(chk < nw)
            for f in range(F_PER):
                plsc.store_scatter(accs[f], [r16], news[f], mask=pends[f])
            for f in range(F_PER):
                chk = plsc.load_gather(accs[f], [r16])
                unresolved = unresolved | (pends[f] & (chk < news[f]))
            return unresolved
        unresolved = lax.fori_loop(
            0, NB, fast_batch, jnp.zeros((16,), jnp.bool_), unroll=2)

        @pl.when(jnp.any(unresolved))
        def _slow():
            def slow_batch(b, c2):
                s16 = s_buf[pl.ds(b * 16, 16)]
                r16 = r_buf[pl.ds(b * 16, 16)]
                news, pends = [], []
                for f in range(F_PER):
                    hv = plsc.load_gather(h_v, [s16 + (f * N_NODES)])
                    cur = plsc.load_gather(accs[f], [r16])
                    nw = jnp.maximum(cur, hv)
                    plsc.store_scatter(accs[f], [r16], nw)
                    chk = plsc.load_gather(accs[f], [r16])
                    news.append(nw)
                    pends.append(chk < nw)

                def cond(ps):
                    return jnp.any(ps[0] | ps[1] | ps[2] | ps[3])

                def retry(ps):
                    out_ps = []
                    for f in range(F_PER):
                        plsc.store_scatter(accs[f], [r16], news[f],
                                           mask=ps[f])
                        chk = plsc.load_gather(accs[f], [r16])
                        out_ps.append(ps[f] & (chk < news[f]))
                    return tuple(out_ps)

                lax.while_loop(cond, retry, tuple(pends))
                return c2
            lax.fori_loop(0, NB, slow_batch, 0)
        return c
    lax.fori_loop(0, NCHUNKS, chunk_body, 0)

    for f in range(F_PER):
        pltpu.sync_copy(accs[f], out_hbm.at[pl.ds(base + f * N_NODES, N_NODES)])


_segmax = functools.partial(
    pl.kernel,
    mesh=plsc.VectorSubcoreMesh(core_axis_name="c", subcore_axis_name="s",
                                num_cores=NC, num_subcores=NS),
    out_type=jax.ShapeDtypeStruct((D * N_NODES,), jnp.float32),
    compiler_params=pltpu.CompilerParams(needs_layout_passes=False),
    scratch_types=[
        pltpu.VMEM((SEG,), jnp.float32),
        pltpu.VMEM((N_NODES,), jnp.float32),
        pltpu.VMEM((N_NODES,), jnp.float32),
        pltpu.VMEM((N_NODES,), jnp.float32),
        pltpu.VMEM((N_NODES,), jnp.float32),
        pltpu.VMEM((CHUNK,), jnp.int32),
        pltpu.VMEM((CHUNK,), jnp.int32),
    ],
)(_segmax_body)


def kernel(node_features, senders, receivers, W1, b1, W2, b2):
    h_T = pl.pallas_call(
        _mm1_body,
        out_shape=jax.ShapeDtypeStruct((D, N_NODES), jnp.float32),
    )(W1, node_features, b1.reshape(D, 1))

    agg_flat = _segmax(h_T.reshape(-1),
                       senders.astype(jnp.int32),
                       receivers.astype(jnp.int32))

    nodes = pl.pallas_call(
        _mm2_body,
        out_shape=jax.ShapeDtypeStruct((N_NODES, D), jnp.float32),
    )(agg_flat.reshape(D, N_NODES), W2, b2.reshape(1, D))
    return nodes
